# baseline (speedup 1.0000x reference)
import functools

import jax
import jax.numpy as jnp
from jax import lax
from jax.experimental import pallas as pl
from jax.experimental.pallas import tpu as pltpu
from jax.experimental.pallas import tpu_sc as plsc

N = 10000
E = 320000
F = 128
D = 16
L = 3
G = 64
DN = 256


def _prep_body(h_ref, wd_ref, ws_ref, bd_ref, td_ref, ts_ref):
    h = h_ref[...]
    td_ref[...] = jnp.dot(h, wd_ref[...], preferred_element_type=jnp.float32) + bd_ref[...]
    ts_ref[...] = jnp.dot(h, ws_ref[...], preferred_element_type=jnp.float32)


def _prep(h, wd, ws, bd):
    blk = 2000
    return pl.pallas_call(
        _prep_body,
        grid=(N // blk,),
        in_specs=[
            pl.BlockSpec((blk, F), lambda i: (i, 0)),
            pl.BlockSpec((F, 2 * F), lambda i: (0, 0)),
            pl.BlockSpec((F, 2 * F), lambda i: (0, 0)),
            pl.BlockSpec((1, 2 * F), lambda i: (0, 0)),
        ],
        out_specs=[
            pl.BlockSpec((blk, 2 * F), lambda i: (i, 0)),
            pl.BlockSpec((blk, 2 * F), lambda i: (i, 0)),
        ],
        out_shape=[
            jax.ShapeDtypeStruct((N, 2 * F), jnp.float32),
            jax.ShapeDtypeStruct((N, 2 * F), jnp.float32),
        ],
    )(h, wd, ws, bd)


def _eproj_body(e_ref, w_ref, r_ref):
    r_ref[0] = jnp.dot(e_ref[...], w_ref[0], preferred_element_type=jnp.float32)


def _eproj(edge_attr, we):
    blk = 16000
    return pl.pallas_call(
        _eproj_body,
        grid=(L, E // blk),
        in_specs=[
            pl.BlockSpec((blk, D), lambda l, i: (i, 0)),
            pl.BlockSpec((1, D, 2 * F), lambda l, i: (l, 0, 0)),
        ],
        out_specs=pl.BlockSpec((1, blk, 2 * F), lambda l, i: (l, i, 0)),
        out_shape=jax.ShapeDtypeStruct((L, E, 2 * F), jnp.float32),
    )(edge_attr, we)


def _post_body(p_ref, h_ref, g_ref, b_ref, o_ref):
    agg = p_ref[0] + p_ref[1]
    mu = jnp.mean(agg, axis=0, keepdims=True)
    cent = agg - mu
    var = jnp.mean(cent * cent, axis=0, keepdims=True)
    scale = g_ref[...] * lax.rsqrt(var + 1e-5)
    o_ref[...] = cent * scale + b_ref[...] + h_ref[...]


def _post(partials, h, gamma, beta):
    return pl.pallas_call(
        _post_body,
        out_shape=jax.ShapeDtypeStruct((N, F), jnp.float32),
    )(partials, h, gamma.reshape(1, F), beta.reshape(1, F))


def _pool_head_body(h_ref, batch_ref, w1_ref, b1_ref, g2_ref, be2_ref,
                    w2_ref, b2_ref, o_ref):
    h = h_ref[...]
    b = batch_ref[...]
    neg = jnp.float32(-jnp.inf)
    rows = []
    for g in range(G):
        m = (b == g)
        rows.append(jnp.max(jnp.where(m, h, neg), axis=0))
    pooled = jnp.stack(rows)
    d = jnp.dot(pooled, w1_ref[...], preferred_element_type=jnp.float32) + b1_ref[...]
    d = jnp.maximum(d, 0.0)
    mu = jnp.mean(d, axis=0, keepdims=True)
    cent = d - mu
    var = jnp.mean(cent * cent, axis=0, keepdims=True)
    d = cent * (g2_ref[...] * lax.rsqrt(var + 1e-5)) + be2_ref[...]
    out = jnp.dot(d, w2_ref[...], preferred_element_type=jnp.float32) + b2_ref[...]
    o_ref[...] = jax.nn.sigmoid(out)


def _pool_head(h, batch, W1, b1, g2, be2, W2, b2):
    return pl.pallas_call(
        _pool_head_body,
        out_shape=jax.ShapeDtypeStruct((G, 1), jnp.float32),
    )(h, batch.reshape(N, 1), W1, b1.reshape(1, DN), g2.reshape(1, DN),
      be2.reshape(1, DN), W2, b2.reshape(1, 1))


_B = 40
_NTILES = 32
_EPT = E // _NTILES
_NCH = _EPT // _B
_RPS = 624
_RTAIL = N - 16 * _RPS


def _sc_edge_body(td_hbm, ts_hbm, r_hbm, src_hbm, dst_hbm, z_hbm, out_hbm,
                  acc, didx, sidx, tdv, tsv, rv, mv, s1, s2, s3):
    c = lax.axis_index("c")
    s = lax.axis_index("s")
    pltpu.sync_copy(z_hbm.at[pl.ds(s * _RPS, _RPS)],
                    acc.at[pl.ds(s * _RPS, _RPS)])

    @pl.when(s == 0)
    def _zero_tail():
        pltpu.sync_copy(z_hbm.at[pl.ds(16 * _RPS, _RTAIL)],
                        acc.at[pl.ds(16 * _RPS, _RTAIL)])

    plsc.subcore_barrier()
    base = (c * 16 + s) * _EPT

    def chunk(i, carry):
        off = base + i * _B
        pltpu.sync_copy(dst_hbm.at[pl.ds(off, _B)], didx)
        pltpu.sync_copy(src_hbm.at[pl.ds(off, _B)], sidx)
        cp1 = pltpu.async_copy(td_hbm.at[didx], tdv, s1)
        cp2 = pltpu.async_copy(ts_hbm.at[sidx], tsv, s2)
        cp3 = pltpu.async_copy(r_hbm.at[pl.ds(off, _B)], rv, s3)
        cp1.wait()
        cp2.wait()
        cp3.wait()

        def row(r_, rcarry):
            for j in range(8):
                sl = pl.ds(j * 16, 16)
                sh = pl.ds(F + j * 16, 16)
                a = tdv[r_, sl] + tsv[r_, sl] + rv[r_, sl]
                bq = tdv[r_, sh] + tsv[r_, sh] + rv[r_, sh]
                u = jnp.exp(-jnp.abs(bq))
                w = u / (u + 2.0)
                w2 = w * w
                p = 1.0 + w2 * (0.333333333 + w2 * (0.2 + w2 * (0.142857143 + w2 * 0.111111111)))
                sp = jnp.maximum(bq, 0.0) + (2.0 * w) * p
                mv[r_, sl] = sp / (1.0 + jnp.exp(-a))
            return rcarry

        lax.fori_loop(0, _B, row, 0)
        pltpu.sync_copy(mv, acc.at[didx], add=True)
        return carry

    lax.fori_loop(0, _NCH, chunk, 0)
    plsc.subcore_barrier()
    pltpu.sync_copy(acc.at[pl.ds(s * _RPS, _RPS)],
                    out_hbm.at[c, pl.ds(s * _RPS, _RPS)])

    @pl.when(s == 0)
    def _out_tail():
        pltpu.sync_copy(acc.at[pl.ds(16 * _RPS, _RTAIL)],
                        out_hbm.at[c, pl.ds(16 * _RPS, _RTAIL)])


@functools.partial(
    pl.kernel,
    out_type=jax.ShapeDtypeStruct((2, N, F), jnp.float32),
    mesh=plsc.VectorSubcoreMesh(core_axis_name="c", subcore_axis_name="s"),
    scratch_types=[
        pltpu.VMEM_SHARED((N, F), jnp.float32),
        pltpu.VMEM((_B,), jnp.int32),
        pltpu.VMEM((_B,), jnp.int32),
        pltpu.VMEM((_B, 2 * F), jnp.float32),
        pltpu.VMEM((_B, 2 * F), jnp.float32),
        pltpu.VMEM((_B, 2 * F), jnp.float32),
        pltpu.VMEM((_B, F), jnp.float32),
        pltpu.SemaphoreType.DMA,
        pltpu.SemaphoreType.DMA,
        pltpu.SemaphoreType.DMA,
    ],
)
def _edge_stage_sc(td_hbm, ts_hbm, r_hbm, src_hbm, dst_hbm, z_hbm, out_hbm,
                   acc, didx, sidx, tdv, tsv, rv, mv, s1, s2, s3):
    _sc_edge_body(td_hbm, ts_hbm, r_hbm, src_hbm, dst_hbm, z_hbm, out_hbm,
                  acc, didx, sidx, tdv, tsv, rv, mv, s1, s2, s3)


def sc_test_fn(td, ts, r, src, dst, z):
    return _edge_stage_sc(td, ts, r, src, dst, z)


def sc_test_args():
    import numpy as _np
    return (_np.zeros((N, 2 * F), _np.float32),
            _np.zeros((N, 2 * F), _np.float32),
            _np.zeros((E, 2 * F), _np.float32),
            _np.zeros((E,), _np.int32),
            _np.zeros((E,), _np.int32),
            _np.zeros((N, F), _np.float32))


def kernel(x, edge_attr, Wf, bf, Ws, bs, gamma, beta, W1, b1, g2, be2, W2, b2,
           edge_index, batch):
    src = edge_index[0]
    dst = edge_index[1]

    wd = jnp.concatenate([Wf[:, :F, :], Ws[:, :F, :]], axis=2)
    wsrc = jnp.concatenate([Wf[:, F:2 * F, :], Ws[:, F:2 * F, :]], axis=2)
    we = jnp.concatenate([Wf[:, 2 * F:, :], Ws[:, 2 * F:, :]], axis=2)
    bd = jnp.concatenate([bf, bs], axis=1)

    r_all = _eproj(edge_attr, we)
    zeros = jnp.zeros((N, F), jnp.float32)

    h = x
    for l in range(L):
        td, tsx = _prep(h, wd[l], wsrc[l], bd[l].reshape(1, 2 * F))
        partials = _edge_stage_sc(td, tsx, r_all[l], src, dst, zeros)
        h = _post(partials, h, gamma[l], beta[l])

    return _pool_head(h, batch, W1, b1, g2, be2, W2, b2)


# speedup vs baseline: 1.0987x; 1.0987x over previous
import functools

import jax
import jax.numpy as jnp
from jax import lax
from jax.experimental import pallas as pl
from jax.experimental.pallas import tpu as pltpu
from jax.experimental.pallas import tpu_sc as plsc

N = 10000
E = 320000
F = 128
D = 16
L = 3
G = 64
DN = 256


def _prep_body(h_ref, wd_ref, ws_ref, bd_ref, td_ref, ts_ref):
    h = h_ref[...]
    td_ref[...] = jnp.dot(h, wd_ref[...], preferred_element_type=jnp.float32) + bd_ref[...]
    ts_ref[...] = jnp.dot(h, ws_ref[...], preferred_element_type=jnp.float32)


def _prep(h, wd, ws, bd):
    blk = 2000
    return pl.pallas_call(
        _prep_body,
        grid=(N // blk,),
        in_specs=[
            pl.BlockSpec((blk, F), lambda i: (i, 0)),
            pl.BlockSpec((F, 2 * F), lambda i: (0, 0)),
            pl.BlockSpec((F, 2 * F), lambda i: (0, 0)),
            pl.BlockSpec((1, 2 * F), lambda i: (0, 0)),
        ],
        out_specs=[
            pl.BlockSpec((blk, 2 * F), lambda i: (i, 0)),
            pl.BlockSpec((blk, 2 * F), lambda i: (i, 0)),
        ],
        out_shape=[
            jax.ShapeDtypeStruct((N, 2 * F), jnp.float32),
            jax.ShapeDtypeStruct((N, 2 * F), jnp.float32),
        ],
    )(h, wd, ws, bd)


def _eproj_body(e_ref, w_ref, r_ref):
    r_ref[0] = jnp.dot(e_ref[...], w_ref[0], preferred_element_type=jnp.float32)


def _eproj(edge_attr, we):
    ep = edge_attr.shape[0]
    blk = 5536
    return pl.pallas_call(
        _eproj_body,
        grid=(L, ep // blk),
        in_specs=[
            pl.BlockSpec((blk, D), lambda l, i: (i, 0)),
            pl.BlockSpec((1, D, 2 * F), lambda l, i: (l, 0, 0)),
        ],
        out_specs=pl.BlockSpec((1, blk, 2 * F), lambda l, i: (l, i, 0)),
        out_shape=jax.ShapeDtypeStruct((L, ep, 2 * F), jnp.float32),
    )(edge_attr, we)


def _post_body(p_ref, h_ref, g_ref, b_ref, o_ref):
    agg = p_ref[0] + p_ref[1]
    mu = jnp.mean(agg, axis=0, keepdims=True)
    cent = agg - mu
    var = jnp.mean(cent * cent, axis=0, keepdims=True)
    scale = g_ref[...] * lax.rsqrt(var + 1e-5)
    o_ref[...] = cent * scale + b_ref[...] + h_ref[...]


def _post(partials, h, gamma, beta):
    return pl.pallas_call(
        _post_body,
        out_shape=jax.ShapeDtypeStruct((N, F), jnp.float32),
    )(partials, h, gamma.reshape(1, F), beta.reshape(1, F))


def _pool_head_body(h_ref, batch_ref, w1_ref, b1_ref, g2_ref, be2_ref,
                    w2_ref, b2_ref, o_ref):
    h = h_ref[...]
    b = batch_ref[...]
    neg = jnp.float32(-jnp.inf)
    rows = []
    for g in range(G):
        m = (b == g)
        rows.append(jnp.max(jnp.where(m, h, neg), axis=0))
    pooled = jnp.stack(rows)
    d = jnp.dot(pooled, w1_ref[...], preferred_element_type=jnp.float32) + b1_ref[...]
    d = jnp.maximum(d, 0.0)
    mu = jnp.mean(d, axis=0, keepdims=True)
    cent = d - mu
    var = jnp.mean(cent * cent, axis=0, keepdims=True)
    d = cent * (g2_ref[...] * lax.rsqrt(var + 1e-5)) + be2_ref[...]
    out = jnp.dot(d, w2_ref[...], preferred_element_type=jnp.float32) + b2_ref[...]
    o_ref[...] = jax.nn.sigmoid(out)


def _pool_head(h, batch, W1, b1, g2, be2, W2, b2):
    return pl.pallas_call(
        _pool_head_body,
        out_shape=jax.ShapeDtypeStruct((G, 1), jnp.float32),
    )(h, batch.reshape(N, 1), W1, b1.reshape(1, DN), g2.reshape(1, DN),
      be2.reshape(1, DN), W2, b2.reshape(1, 1))


_B = 24
_S = 8
_NSUP = 54
_C = _S * _NSUP
_EPT = _B * _C
_EP = 32 * _EPT
_EPP = _EP + 2 * _S * _B
_IR = _EPP // _B
_NP = N + 8


def _sc_edge_body(td_hbm, ts_hbm, r_hbm, d2_hbm, s2_hbm, z_hbm, out_hbm,
                  acc,
                  dib0, dib1, sib0, sib1,
                  tdv0, tdv1, tsv0, tsv1, rv0, rv1, mv0, mv1,
                  g00, g01, g02, g10, g11, g12):
    c = lax.axis_index("c")
    s = lax.axis_index("s")
    wid = c * 16 + s
    brow = wid * _C

    pltpu.sync_copy(z_hbm.at[pl.ds(0, 624)], acc.at[pl.ds(s * 624, 624)])

    @pl.when(s == 0)
    def _zero_tail():
        pltpu.sync_copy(z_hbm.at[pl.ds(0, 24)], acc.at[pl.ds(9984, 24)])

    plsc.subcore_barrier()

    dib = (dib0, dib1)
    sib = (sib0, sib1)
    tdv = (tdv0, tdv1)
    tsv = (tsv0, tsv1)
    rv = (rv0, rv1)
    mv = (mv0, mv1)
    gsem = ((g00, g01, g02), (g10, g11, g12))

    def idx_load(p, j):
        row = brow + j * _S
        pltpu.sync_copy(d2_hbm.at[pl.ds(row, _S)], dib[p])
        pltpu.sync_copy(s2_hbm.at[pl.ds(row, _S)], sib[p])

    def fire_gathers(slot, p, k, i):
        off = (brow + i) * _B
        pltpu.async_copy(td_hbm.at[dib[p].at[k]], tdv[slot], gsem[slot][0])
        pltpu.async_copy(ts_hbm.at[sib[p].at[k]], tsv[slot], gsem[slot][1])
        pltpu.async_copy(r_hbm.at[pl.ds(off, _B)], rv[slot], gsem[slot][2])

    def wait_gathers(slot, p, k):
        pltpu.make_async_copy(td_hbm.at[dib[p].at[k]], tdv[slot], gsem[slot][0]).wait()
        pltpu.make_async_copy(ts_hbm.at[sib[p].at[k]], tsv[slot], gsem[slot][1]).wait()
        pltpu.make_async_copy(r_hbm.at[pl.ds(0, _B)], rv[slot], gsem[slot][2]).wait()

    def compute(slot):
        td_, ts_, r_, m_ = tdv[slot], tsv[slot], rv[slot], mv[slot]

        def row(rr, carry):
            for g in range(8):
                sl = pl.ds(g * 16, 16)
                sh = pl.ds(F + g * 16, 16)
                a = td_[rr, sl] + ts_[rr, sl] + r_[rr, sl]
                bq = td_[rr, sh] + ts_[rr, sh] + r_[rr, sh]
                u = jnp.exp(-jnp.abs(bq))
                w = u / (u + 2.0)
                w2 = w * w
                p_ = 1.0 + w2 * (0.333333333 + w2 * (0.2 + w2 * (0.142857143 + w2 * 0.111111111)))
                sp = jnp.maximum(bq, 0.0) + (2.0 * w) * p_
                m_[rr, sl] = sp / (1.0 + jnp.exp(-a))
            return carry

        lax.fori_loop(0, _B, row, 0)

    idx_load(0, 0)
    fire_gathers(0, 0, 0, 0)
    fire_gathers(1, 0, 1, 1)
    idx_load(1, 1)

    def dbody(jj, carry):
        for half in range(2):
            j = 2 * jj + half
            p = half
            q = 1 - half
            for k in range(_S):
                slot = k % 2
                i = j * _S + k
                wait_gathers(slot, p, k)
                compute(slot)
                pltpu.sync_copy(mv[slot], acc.at[dib[p].at[k]], add=True)
                if k == 2:
                    idx_load(q, j + 1)
                if k < _S - 2:
                    fire_gathers(slot, p, k + 2, i + 2)
                else:
                    fire_gathers(slot, q, k + 2 - _S, i + 2)
        return carry

    lax.fori_loop(0, _NSUP // 2, dbody, 0)

    wait_gathers(0, 0, 0)
    wait_gathers(1, 0, 1)

    plsc.subcore_barrier()
    pltpu.sync_copy(acc.at[pl.ds(s * 624, 624)],
                    out_hbm.at[c, pl.ds(s * 624, 624)])

    @pl.when(s == 0)
    def _out_tail():
        pltpu.sync_copy(acc.at[pl.ds(9984, 16)],
                        out_hbm.at[c, pl.ds(9984, 16)])


@functools.partial(
    pl.kernel,
    out_type=jax.ShapeDtypeStruct((2, N, F), jnp.float32),
    mesh=plsc.VectorSubcoreMesh(core_axis_name="c", subcore_axis_name="s"),
    scratch_types=[
        pltpu.VMEM_SHARED((_NP, F), jnp.float32),
        pltpu.VMEM((_S, _B), jnp.int32),
        pltpu.VMEM((_S, _B), jnp.int32),
        pltpu.VMEM((_S, _B), jnp.int32),
        pltpu.VMEM((_S, _B), jnp.int32),
        pltpu.VMEM((_B, 2 * F), jnp.float32),
        pltpu.VMEM((_B, 2 * F), jnp.float32),
        pltpu.VMEM((_B, 2 * F), jnp.float32),
        pltpu.VMEM((_B, 2 * F), jnp.float32),
        pltpu.VMEM((_B, 2 * F), jnp.float32),
        pltpu.VMEM((_B, 2 * F), jnp.float32),
        pltpu.VMEM((_B, F), jnp.float32),
        pltpu.VMEM((_B, F), jnp.float32),
        pltpu.SemaphoreType.DMA,
        pltpu.SemaphoreType.DMA,
        pltpu.SemaphoreType.DMA,
        pltpu.SemaphoreType.DMA,
        pltpu.SemaphoreType.DMA,
        pltpu.SemaphoreType.DMA,
    ],
)
def _edge_stage_sc(td_hbm, ts_hbm, r_hbm, d2_hbm, s2_hbm, z_hbm, out_hbm,
                   acc,
                   dib0, dib1, sib0, sib1,
                   tdv0, tdv1, tsv0, tsv1, rv0, rv1, mv0, mv1,
                   g00, g01, g02, g10, g11, g12):
    _sc_edge_body(td_hbm, ts_hbm, r_hbm, d2_hbm, s2_hbm, z_hbm, out_hbm,
                  acc,
                  dib0, dib1, sib0, sib1,
                  tdv0, tdv1, tsv0, tsv1, rv0, rv1, mv0, mv1,
                  g00, g01, g02, g10, g11, g12)


def sc_test_fn(td, ts, r, d2, s2, z):
    return _edge_stage_sc(td, ts, r, d2, s2, z)


def sc_test_args():
    import numpy as _np
    return (_np.zeros((_NP, 2 * F), _np.float32),
            _np.zeros((_NP, 2 * F), _np.float32),
            _np.zeros((_EPP, 2 * F), _np.float32),
            _np.zeros((_IR, _B), _np.int32),
            _np.zeros((_IR, _B), _np.int32),
            _np.zeros((632, F), _np.float32))


def kernel(x, edge_attr, Wf, bf, Ws, bs, gamma, beta, W1, b1, g2, be2, W2, b2,
           edge_index, batch):
    src = edge_index[0]
    dst = edge_index[1]

    wd = jnp.concatenate([Wf[:, :F, :], Ws[:, :F, :]], axis=2)
    wsrc = jnp.concatenate([Wf[:, F:2 * F, :], Ws[:, F:2 * F, :]], axis=2)
    we = jnp.concatenate([Wf[:, 2 * F:, :], Ws[:, 2 * F:, :]], axis=2)
    bd = jnp.concatenate([bf, bs], axis=1)

    pad = _EPP - E
    dst2 = jnp.concatenate([dst, jnp.full((pad,), N, jnp.int32)]).reshape(_IR, _B)
    src2 = jnp.concatenate([src, jnp.full((pad,), N, jnp.int32)]).reshape(_IR, _B)
    ea_p = jnp.concatenate([edge_attr, jnp.zeros((pad, D), jnp.float32)])
    r_all = _eproj(ea_p, we)
    zeros = jnp.zeros((632, F), jnp.float32)

    h = x
    for l in range(L):
        td, tsx = _prep(h, wd[l], wsrc[l], bd[l].reshape(1, 2 * F))
        tdp = jnp.pad(td, ((0, _NP - N), (0, 0)))
        tsp = jnp.pad(tsx, ((0, _NP - N), (0, 0)))
        partials = _edge_stage_sc(tdp, tsp, r_all[l], dst2, src2, zeros)
        h = _post(partials, h, gamma[l], beta[l])

    return _pool_head(h, batch, W1, b1, g2, be2, W2, b2)


# speedup vs baseline: 1.2371x; 1.1260x over previous
import functools

import jax
import jax.numpy as jnp
from jax import lax
from jax.experimental import pallas as pl
from jax.experimental.pallas import tpu as pltpu
from jax.experimental.pallas import tpu_sc as plsc

N = 10000
E = 320000
F = 128
D = 16
L = 3
G = 64
DN = 256


def _prep_body(h_ref, wd_ref, ws_ref, bd_ref, td_ref, ts_ref):
    h = h_ref[...]
    td_ref[...] = jnp.dot(h, wd_ref[...], preferred_element_type=jnp.float32) + bd_ref[...]
    ts_ref[...] = jnp.dot(h, ws_ref[...], preferred_element_type=jnp.float32)


def _prep(h, wd, ws, bd):
    blk = 2000
    return pl.pallas_call(
        _prep_body,
        grid=(N // blk,),
        in_specs=[
            pl.BlockSpec((blk, F), lambda i: (i, 0)),
            pl.BlockSpec((F, 2 * F), lambda i: (0, 0)),
            pl.BlockSpec((F, 2 * F), lambda i: (0, 0)),
            pl.BlockSpec((1, 2 * F), lambda i: (0, 0)),
        ],
        out_specs=[
            pl.BlockSpec((blk, 2 * F), lambda i: (i, 0)),
            pl.BlockSpec((blk, 2 * F), lambda i: (i, 0)),
        ],
        out_shape=[
            jax.ShapeDtypeStruct((N, 2 * F), jnp.float32),
            jax.ShapeDtypeStruct((N, 2 * F), jnp.float32),
        ],
    )(h, wd, ws, bd)


def _eproj_body(e_ref, w_ref, r_ref):
    r_ref[0] = jnp.dot(e_ref[...], w_ref[0], preferred_element_type=jnp.float32)


def _eproj(edge_attr, we):
    ep = edge_attr.shape[0]
    blk = 6928
    return pl.pallas_call(
        _eproj_body,
        grid=(L, ep // blk),
        in_specs=[
            pl.BlockSpec((blk, D), lambda l, i: (i, 0)),
            pl.BlockSpec((1, D, 2 * F), lambda l, i: (l, 0, 0)),
        ],
        out_specs=pl.BlockSpec((1, blk, 2 * F), lambda l, i: (l, i, 0)),
        out_shape=jax.ShapeDtypeStruct((L, ep, 2 * F), jnp.float32),
    )(edge_attr, we)


def _post_body(p_ref, h_ref, g_ref, b_ref, o_ref):
    agg = p_ref[0] + p_ref[1]
    mu = jnp.mean(agg, axis=0, keepdims=True)
    cent = agg - mu
    var = jnp.mean(cent * cent, axis=0, keepdims=True)
    scale = g_ref[...] * lax.rsqrt(var + 1e-5)
    o_ref[...] = cent * scale + b_ref[...] + h_ref[...]


def _post(partials, h, gamma, beta):
    return pl.pallas_call(
        _post_body,
        out_shape=jax.ShapeDtypeStruct((N, F), jnp.float32),
    )(partials, h, gamma.reshape(1, F), beta.reshape(1, F))


def _pool_head_body(h_ref, batch_ref, w1_ref, b1_ref, g2_ref, be2_ref,
                    w2_ref, b2_ref, o_ref):
    h = h_ref[...]
    b = batch_ref[...]
    neg = jnp.float32(-jnp.inf)
    rows = []
    for g in range(G):
        m = (b == g)
        rows.append(jnp.max(jnp.where(m, h, neg), axis=0))
    pooled = jnp.stack(rows)
    d = jnp.dot(pooled, w1_ref[...], preferred_element_type=jnp.float32) + b1_ref[...]
    d = jnp.maximum(d, 0.0)
    mu = jnp.mean(d, axis=0, keepdims=True)
    cent = d - mu
    var = jnp.mean(cent * cent, axis=0, keepdims=True)
    d = cent * (g2_ref[...] * lax.rsqrt(var + 1e-5)) + be2_ref[...]
    out = jnp.dot(d, w2_ref[...], preferred_element_type=jnp.float32) + b2_ref[...]
    o_ref[...] = jax.nn.sigmoid(out)


def _pool_head(h, batch, W1, b1, g2, be2, W2, b2):
    return pl.pallas_call(
        _pool_head_body,
        out_shape=jax.ShapeDtypeStruct((G, 1), jnp.float32),
    )(h, batch.reshape(N, 1), W1, b1.reshape(1, DN), g2.reshape(1, DN),
      be2.reshape(1, DN), W2, b2.reshape(1, 1))


_B = 16
_S = 24
_NSUP = 27
_C = _S * _NSUP
_EPT = _B * _C
_EP = 32 * _EPT
_EPP = _EP + 2 * _S * _B
_IR = _EPP // _B
_SW = 3 * _B
_IRB = _EPP // _SW
_NP = N + 8


def _sc_edge_body(td_hbm, ts_hbm, r_hbm, d2_hbm, s2_hbm, db_hbm, z_hbm,
                  out_hbm,
                  acc,
                  dib, sib, dsb,
                  tdv0, tdv1, tsv0, tsv1, rv0, rv1, mv,
                  g00, g01, g02, g10, g11, g12):
    c = lax.axis_index("c")
    s = lax.axis_index("s")
    wid = c * 16 + s
    brow = wid * _C
    brow2 = wid * (_C // 3)

    pltpu.sync_copy(z_hbm.at[pl.ds(0, 624)], acc.at[pl.ds(s * 624, 624)])

    @pl.when(s == 0)
    def _zero_tail():
        pltpu.sync_copy(z_hbm.at[pl.ds(0, 24)], acc.at[pl.ds(9984, 24)])

    plsc.subcore_barrier()

    tdv = (tdv0, tdv1)
    tsv = (tsv0, tsv1)
    rv = (rv0, rv1)
    gsem = ((g00, g01, g02), (g10, g11, g12))

    def idx_load(p, j):
        row = brow + j * _S
        pltpu.sync_copy(d2_hbm.at[pl.ds(row, _S)], dib.at[p])
        pltpu.sync_copy(s2_hbm.at[pl.ds(row, _S)], sib.at[p])
        pltpu.sync_copy(db_hbm.at[pl.ds(brow2 + j * (_S // 3), _S // 3)],
                        dsb.at[p])

    def fire_gathers(slot, p, k, i):
        off = (brow + i) * _B
        pltpu.async_copy(td_hbm.at[dib.at[p, k]], tdv[slot], gsem[slot][0])
        pltpu.async_copy(ts_hbm.at[sib.at[p, k]], tsv[slot], gsem[slot][1])
        pltpu.async_copy(r_hbm.at[pl.ds(off, _B)], rv[slot], gsem[slot][2])

    def wait_gathers(slot, p, k):
        pltpu.make_async_copy(td_hbm.at[dib.at[p, k]], tdv[slot], gsem[slot][0]).wait()
        pltpu.make_async_copy(ts_hbm.at[sib.at[p, k]], tsv[slot], gsem[slot][1]).wait()
        pltpu.make_async_copy(r_hbm.at[pl.ds(0, _B)], rv[slot], gsem[slot][2]).wait()

    def compute(slot, mrow):
        td_, ts_, r_ = tdv[slot], tsv[slot], rv[slot]

        def row(rr, carry):
            for g in range(8):
                sl = pl.ds(g * 16, 16)
                sh = pl.ds(F + g * 16, 16)
                a = td_[rr, sl] + ts_[rr, sl] + r_[rr, sl]
                bq = td_[rr, sh] + ts_[rr, sh] + r_[rr, sh]
                u = jnp.exp(-jnp.abs(bq))
                lp = u * (0.9999971 + u * (-0.4998254 + u * (0.3307875 + u * (-0.2341725 + u * (0.1481052 + u * (-0.0657691 + u * 0.0140266))))))
                sp = jnp.maximum(bq, 0.0) + lp
                mv[mrow + rr, sl] = sp / (1.0 + jnp.exp(-a))
            return carry

        lax.fori_loop(0, _B, row, 0)

    idx_load(0, 0)
    fire_gathers(0, 0, 0, 0)
    fire_gathers(1, 0, 1, 1)
    idx_load(1, 1)

    def pbody(i2, carry):
        for sub in range(2):
            i = 2 * i2 + sub
            j = i // _S
            k = i % _S
            p = lax.rem(j, 2)
            wait_gathers(sub, p, k)
            compute(sub, lax.rem(k, 3) * _B)

            @pl.when(lax.rem(k, 3) == 2)
            def _sc():
                pltpu.sync_copy(mv, acc.at[dsb.at[p, k // 3]], add=True)

            @pl.when(k == 2)
            def _il():
                idx_load(1 - p, j + 1)

            i_n = i + 2
            fire_gathers(sub, lax.rem(i_n // _S, 2), i_n % _S, i_n)
        return carry

    lax.fori_loop(0, _C // 2, pbody, 0)

    wait_gathers(0, 0, 0)
    wait_gathers(1, 0, 1)

    plsc.subcore_barrier()
    pltpu.sync_copy(acc.at[pl.ds(s * 624, 624)],
                    out_hbm.at[c, pl.ds(s * 624, 624)])

    @pl.when(s == 0)
    def _out_tail():
        pltpu.sync_copy(acc.at[pl.ds(9984, 16)],
                        out_hbm.at[c, pl.ds(9984, 16)])


@functools.partial(
    pl.kernel,
    out_type=jax.ShapeDtypeStruct((2, N, F), jnp.float32),
    mesh=plsc.VectorSubcoreMesh(core_axis_name="c", subcore_axis_name="s"),
    scratch_types=[
        pltpu.VMEM_SHARED((_NP, F), jnp.float32),
        pltpu.VMEM((2, _S, _B), jnp.int32),
        pltpu.VMEM((2, _S, _B), jnp.int32),
        pltpu.VMEM((2, _S // 3, _SW), jnp.int32),
        pltpu.VMEM((_B, 2 * F), jnp.float32),
        pltpu.VMEM((_B, 2 * F), jnp.float32),
        pltpu.VMEM((_B, 2 * F), jnp.float32),
        pltpu.VMEM((_B, 2 * F), jnp.float32),
        pltpu.VMEM((_B, 2 * F), jnp.float32),
        pltpu.VMEM((_B, 2 * F), jnp.float32),
        pltpu.VMEM((_SW, F), jnp.float32),
        pltpu.SemaphoreType.DMA,
        pltpu.SemaphoreType.DMA,
        pltpu.SemaphoreType.DMA,
        pltpu.SemaphoreType.DMA,
        pltpu.SemaphoreType.DMA,
        pltpu.SemaphoreType.DMA,
    ],
)
def _edge_stage_sc(td_hbm, ts_hbm, r_hbm, d2_hbm, s2_hbm, db_hbm, z_hbm,
                   out_hbm,
                   acc,
                   dib, sib, dsb,
                   tdv0, tdv1, tsv0, tsv1, rv0, rv1, mv,
                   g00, g01, g02, g10, g11, g12):
    _sc_edge_body(td_hbm, ts_hbm, r_hbm, d2_hbm, s2_hbm, db_hbm, z_hbm,
                  out_hbm,
                  acc,
                  dib, sib, dsb,
                  tdv0, tdv1, tsv0, tsv1, rv0, rv1, mv,
                  g00, g01, g02, g10, g11, g12)


def sc_test_fn(td, ts, r, d2, s2, db, z):
    return _edge_stage_sc(td, ts, r, d2, s2, db, z)


def sc_test_args():
    import numpy as _np
    return (_np.zeros((_NP, 2 * F), _np.float32),
            _np.zeros((_NP, 2 * F), _np.float32),
            _np.zeros((_EPP, 2 * F), _np.float32),
            _np.zeros((_IR, _B), _np.int32),
            _np.zeros((_IR, _B), _np.int32),
            _np.zeros((_IRB, _SW), _np.int32),
            _np.zeros((632, F), _np.float32))


def kernel(x, edge_attr, Wf, bf, Ws, bs, gamma, beta, W1, b1, g2, be2, W2, b2,
           edge_index, batch):
    src = edge_index[0]
    dst = edge_index[1]

    wd = jnp.concatenate([Wf[:, :F, :], Ws[:, :F, :]], axis=2)
    wsrc = jnp.concatenate([Wf[:, F:2 * F, :], Ws[:, F:2 * F, :]], axis=2)
    we = jnp.concatenate([Wf[:, 2 * F:, :], Ws[:, 2 * F:, :]], axis=2)
    bd = jnp.concatenate([bf, bs], axis=1)

    pad = _EPP - E
    dst_p = jnp.concatenate([dst, jnp.full((pad,), N, jnp.int32)])
    dst2 = dst_p.reshape(_IR, _B)
    dstb = dst_p.reshape(_IRB, _SW)
    src2 = jnp.concatenate([src, jnp.full((pad,), N, jnp.int32)]).reshape(_IR, _B)
    ea_p = jnp.concatenate([edge_attr, jnp.zeros((pad, D), jnp.float32)])
    r_all = _eproj(ea_p, we)
    zeros = jnp.zeros((632, F), jnp.float32)

    h = x
    for l in range(L):
        td, tsx = _prep(h, wd[l], wsrc[l], bd[l].reshape(1, 2 * F))
        tdp = jnp.pad(td, ((0, _NP - N), (0, 0)))
        tsp = jnp.pad(tsx, ((0, _NP - N), (0, 0)))
        partials = _edge_stage_sc(tdp, tsp, r_all[l], dst2, src2, dstb, zeros)
        h = _post(partials, h, gamma[l], beta[l])

    return _pool_head(h, batch, W1, b1, g2, be2, W2, b2)


# speedup vs baseline: 1.2388x; 1.0014x over previous
import functools

import jax
import jax.numpy as jnp
from jax import lax
from jax.experimental import pallas as pl
from jax.experimental.pallas import tpu as pltpu
from jax.experimental.pallas import tpu_sc as plsc

N = 10000
E = 320000
F = 128
D = 16
L = 3
G = 64
DN = 256


def _prep_body(h_ref, wd_ref, ws_ref, bd_ref, td_ref, ts_ref):
    h = h_ref[...]
    td_ref[...] = jnp.dot(h, wd_ref[...], preferred_element_type=jnp.float32) + bd_ref[...]
    ts_ref[...] = jnp.dot(h, ws_ref[...], preferred_element_type=jnp.float32)


def _prep(h, wd, ws, bd):
    blk = 2000
    return pl.pallas_call(
        _prep_body,
        grid=(N // blk,),
        in_specs=[
            pl.BlockSpec((blk, F), lambda i: (i, 0)),
            pl.BlockSpec((F, 2 * F), lambda i: (0, 0)),
            pl.BlockSpec((F, 2 * F), lambda i: (0, 0)),
            pl.BlockSpec((1, 2 * F), lambda i: (0, 0)),
        ],
        out_specs=[
            pl.BlockSpec((blk, 2 * F), lambda i: (i, 0)),
            pl.BlockSpec((blk, 2 * F), lambda i: (i, 0)),
        ],
        out_shape=[
            jax.ShapeDtypeStruct((N, 2 * F), jnp.float32),
            jax.ShapeDtypeStruct((N, 2 * F), jnp.float32),
        ],
    )(h, wd, ws, bd)


def _eproj_body(e_ref, w_ref, r_ref):
    r_ref[0] = jnp.dot(e_ref[...], w_ref[0], preferred_element_type=jnp.float32)


def _eproj(edge_attr, we):
    ep = edge_attr.shape[0]
    blk = 6928
    return pl.pallas_call(
        _eproj_body,
        grid=(L, ep // blk),
        in_specs=[
            pl.BlockSpec((blk, D), lambda l, i: (i, 0)),
            pl.BlockSpec((1, D, 2 * F), lambda l, i: (l, 0, 0)),
        ],
        out_specs=pl.BlockSpec((1, blk, 2 * F), lambda l, i: (l, i, 0)),
        out_shape=jax.ShapeDtypeStruct((L, ep, 2 * F), jnp.float32),
    )(edge_attr, we)


def _post_body(p_ref, h_ref, g_ref, b_ref, o_ref):
    agg = p_ref[0] + p_ref[1]
    mu = jnp.mean(agg, axis=0, keepdims=True)
    cent = agg - mu
    var = jnp.mean(cent * cent, axis=0, keepdims=True)
    scale = g_ref[...] * lax.rsqrt(var + 1e-5)
    o_ref[...] = cent * scale + b_ref[...] + h_ref[...]


def _post(partials, h, gamma, beta):
    return pl.pallas_call(
        _post_body,
        out_shape=jax.ShapeDtypeStruct((N, F), jnp.float32),
    )(partials, h, gamma.reshape(1, F), beta.reshape(1, F))


def _pool_head_body(h_ref, batch_ref, w1_ref, b1_ref, g2_ref, be2_ref,
                    w2_ref, b2_ref, o_ref):
    h = h_ref[...]
    b = batch_ref[...]
    neg = jnp.float32(-jnp.inf)
    rows = []
    for g in range(G):
        m = (b == g)
        rows.append(jnp.max(jnp.where(m, h, neg), axis=0))
    pooled = jnp.stack(rows)
    d = jnp.dot(pooled, w1_ref[...], preferred_element_type=jnp.float32) + b1_ref[...]
    d = jnp.maximum(d, 0.0)
    mu = jnp.mean(d, axis=0, keepdims=True)
    cent = d - mu
    var = jnp.mean(cent * cent, axis=0, keepdims=True)
    d = cent * (g2_ref[...] * lax.rsqrt(var + 1e-5)) + be2_ref[...]
    out = jnp.dot(d, w2_ref[...], preferred_element_type=jnp.float32) + b2_ref[...]
    o_ref[...] = jax.nn.sigmoid(out)


def _pool_head(h, batch, W1, b1, g2, be2, W2, b2):
    return pl.pallas_call(
        _pool_head_body,
        out_shape=jax.ShapeDtypeStruct((G, 1), jnp.float32),
    )(h, batch.reshape(N, 1), W1, b1.reshape(1, DN), g2.reshape(1, DN),
      be2.reshape(1, DN), W2, b2.reshape(1, 1))


_B = 16
_S = 24
_NSUP = 27
_C = _S * _NSUP
_EPT = _B * _C
_EP = 32 * _EPT
_EPP = _EP + 2 * _S * _B
_IR = _EPP // _B
_SW = 3 * _B
_IRB = _EPP // _SW
_NP = N + 8


def _sc_edge_body(td_hbm, ts_hbm, r_hbm, d2_hbm, s2_hbm, db_hbm, z_hbm,
                  out_hbm,
                  acc,
                  dib, sib, dsb,
                  tdv0, tdv1, tsv0, tsv1, rv0, rv1, mv,
                  g00, g01, g02, g10, g11, g12):
    c = lax.axis_index("c")
    s = lax.axis_index("s")
    wid = c * 16 + s
    brow = wid * _C
    brow2 = wid * (_C // 3)

    pltpu.sync_copy(z_hbm.at[pl.ds(0, 624)], acc.at[pl.ds(s * 624, 624)])

    @pl.when(s == 0)
    def _zero_tail():
        pltpu.sync_copy(z_hbm.at[pl.ds(0, 24)], acc.at[pl.ds(9984, 24)])

    plsc.subcore_barrier()

    tdv = (tdv0, tdv1)
    tsv = (tsv0, tsv1)
    rv = (rv0, rv1)
    gsem = ((g00, g01, g02), (g10, g11, g12))

    def idx_load(p, j):
        row = brow + j * _S
        pltpu.sync_copy(d2_hbm.at[pl.ds(row, _S)], dib.at[p])
        pltpu.sync_copy(s2_hbm.at[pl.ds(row, _S)], sib.at[p])
        pltpu.sync_copy(db_hbm.at[pl.ds(brow2 + j * (_S // 3), _S // 3)],
                        dsb.at[p])

    def fire_gathers(slot, p, k, i):
        off = (brow + i) * _B
        pltpu.async_copy(td_hbm.at[dib.at[p, k]], tdv[slot], gsem[slot][0])
        pltpu.async_copy(ts_hbm.at[sib.at[p, k]], tsv[slot], gsem[slot][1])
        pltpu.async_copy(r_hbm.at[pl.ds(off, _B)], rv[slot], gsem[slot][2])

    def wait_gathers(slot, p, k):
        pltpu.make_async_copy(td_hbm.at[dib.at[p, k]], tdv[slot], gsem[slot][0]).wait()
        pltpu.make_async_copy(ts_hbm.at[sib.at[p, k]], tsv[slot], gsem[slot][1]).wait()
        pltpu.make_async_copy(r_hbm.at[pl.ds(0, _B)], rv[slot], gsem[slot][2]).wait()

    def compute(slot, mrow):
        td_, ts_, r_ = tdv[slot], tsv[slot], rv[slot]

        def row(rr, carry):
            for rsub in range(2):
                rr2 = 2 * rr + rsub
                for g in range(8):
                    sl = pl.ds(g * 16, 16)
                    sh = pl.ds(F + g * 16, 16)
                    a = td_[rr2, sl] + ts_[rr2, sl] + r_[rr2, sl]
                    bq = td_[rr2, sh] + ts_[rr2, sh] + r_[rr2, sh]
                    u = jnp.exp(-jnp.abs(bq))
                    lp = u * (0.9999971 + u * (-0.4998254 + u * (0.3307875 + u * (-0.2341725 + u * (0.1481052 + u * (-0.0657691 + u * 0.0140266))))))
                    sp = jnp.maximum(bq, 0.0) + lp
                    mv[mrow + rr2, sl] = sp / (1.0 + jnp.exp(-a))
            return carry

        lax.fori_loop(0, _B // 2, row, 0)

    idx_load(0, 0)
    fire_gathers(0, 0, 0, 0)
    fire_gathers(1, 0, 1, 1)
    idx_load(1, 1)

    def pbody(i2, carry):
        for sub in range(2):
            i = 2 * i2 + sub
            j = i // _S
            k = i % _S
            p = lax.rem(j, 2)
            wait_gathers(sub, p, k)
            compute(sub, lax.rem(k, 3) * _B)

            @pl.when(lax.rem(k, 3) == 2)
            def _sc():
                pltpu.sync_copy(mv, acc.at[dsb.at[p, k // 3]], add=True)

            @pl.when(k == 2)
            def _il():
                idx_load(1 - p, j + 1)

            i_n = i + 2
            fire_gathers(sub, lax.rem(i_n // _S, 2), i_n % _S, i_n)
        return carry

    lax.fori_loop(0, _C // 2, pbody, 0)

    wait_gathers(0, 0, 0)
    wait_gathers(1, 0, 1)

    plsc.subcore_barrier()
    pltpu.sync_copy(acc.at[pl.ds(s * 624, 624)],
                    out_hbm.at[c, pl.ds(s * 624, 624)])

    @pl.when(s == 0)
    def _out_tail():
        pltpu.sync_copy(acc.at[pl.ds(9984, 16)],
                        out_hbm.at[c, pl.ds(9984, 16)])


@functools.partial(
    pl.kernel,
    out_type=jax.ShapeDtypeStruct((2, N, F), jnp.float32),
    mesh=plsc.VectorSubcoreMesh(core_axis_name="c", subcore_axis_name="s"),
    scratch_types=[
        pltpu.VMEM_SHARED((_NP, F), jnp.float32),
        pltpu.VMEM((2, _S, _B), jnp.int32),
        pltpu.VMEM((2, _S, _B), jnp.int32),
        pltpu.VMEM((2, _S // 3, _SW), jnp.int32),
        pltpu.VMEM((_B, 2 * F), jnp.float32),
        pltpu.VMEM((_B, 2 * F), jnp.float32),
        pltpu.VMEM((_B, 2 * F), jnp.float32),
        pltpu.VMEM((_B, 2 * F), jnp.float32),
        pltpu.VMEM((_B, 2 * F), jnp.float32),
        pltpu.VMEM((_B, 2 * F), jnp.float32),
        pltpu.VMEM((_SW, F), jnp.float32),
        pltpu.SemaphoreType.DMA,
        pltpu.SemaphoreType.DMA,
        pltpu.SemaphoreType.DMA,
        pltpu.SemaphoreType.DMA,
        pltpu.SemaphoreType.DMA,
        pltpu.SemaphoreType.DMA,
    ],
)
def _edge_stage_sc(td_hbm, ts_hbm, r_hbm, d2_hbm, s2_hbm, db_hbm, z_hbm,
                   out_hbm,
                   acc,
                   dib, sib, dsb,
                   tdv0, tdv1, tsv0, tsv1, rv0, rv1, mv,
                   g00, g01, g02, g10, g11, g12):
    _sc_edge_body(td_hbm, ts_hbm, r_hbm, d2_hbm, s2_hbm, db_hbm, z_hbm,
                  out_hbm,
                  acc,
                  dib, sib, dsb,
                  tdv0, tdv1, tsv0, tsv1, rv0, rv1, mv,
                  g00, g01, g02, g10, g11, g12)


def kernel(x, edge_attr, Wf, bf, Ws, bs, gamma, beta, W1, b1, g2, be2, W2, b2,
           edge_index, batch):
    src = edge_index[0]
    dst = edge_index[1]

    wd = jnp.concatenate([Wf[:, :F, :], Ws[:, :F, :]], axis=2)
    wsrc = jnp.concatenate([Wf[:, F:2 * F, :], Ws[:, F:2 * F, :]], axis=2)
    we = jnp.concatenate([Wf[:, 2 * F:, :], Ws[:, 2 * F:, :]], axis=2)
    bd = jnp.concatenate([bf, bs], axis=1)

    pad = _EPP - E
    dst_p = jnp.concatenate([dst, jnp.full((pad,), N, jnp.int32)])
    dst2 = dst_p.reshape(_IR, _B)
    dstb = dst_p.reshape(_IRB, _SW)
    src2 = jnp.concatenate([src, jnp.full((pad,), N, jnp.int32)]).reshape(_IR, _B)
    ea_p = jnp.concatenate([edge_attr, jnp.zeros((pad, D), jnp.float32)])
    r_all = _eproj(ea_p, we)
    zeros = jnp.zeros((632, F), jnp.float32)

    h = x
    for l in range(L):
        td, tsx = _prep(h, wd[l], wsrc[l], bd[l].reshape(1, 2 * F))
        tdp = jnp.pad(td, ((0, _NP - N), (0, 0)))
        tsp = jnp.pad(tsx, ((0, _NP - N), (0, 0)))
        partials = _edge_stage_sc(tdp, tsp, r_all[l], dst2, src2, dstb, zeros)
        h = _post(partials, h, gamma[l], beta[l])

    return _pool_head(h, batch, W1, b1, g2, be2, W2, b2)
